# Initial kernel scaffold; baseline (speedup 1.0000x reference)
import jax
import jax.numpy as jnp
from jax.experimental import pallas as pl


def kernel(x, edge_index, Wl1, Wr1, b1, Wl2, Wr2, b2, Wf1, bf1, Wf2, bf2, Wf3, bf3):
    raise NotImplementedError("write your pallas kernel here")


# speedup vs baseline: 8.3737x; 8.3737x over previous
import functools

import jax
import jax.numpy as jnp
from jax import lax
from jax.experimental import pallas as pl
from jax.experimental.pallas import tpu as pltpu
from jax.experimental.pallas import tpu_sc as plsc

N = 10000
E = 320000
D_IN = 128

NC = 2
NS = 16
NW = NC * NS
K = 128
CH = -(-E // (NW * K))
EPAD = NW * K * CH
ROWS = EPAD // K
NP = 10240
SLAB = NP // NS


def _seg_kernel_body(with_counts, table, srcr, dstr, zrows, zcnt, ones,
                     acc_out, cnt_out, src_idx, dst_idx, rows_v, ones_v,
                     acc_sh, cnt_sh, sem):
    c = lax.axis_index("c")
    s = lax.axis_index("s")
    wid = c * NS + s

    pltpu.sync_copy(zrows, acc_sh.at[pl.ds(s * SLAB, SLAB)])
    if with_counts:
        pltpu.sync_copy(zrows, cnt_sh.at[pl.ds(s * SLAB, SLAB)])
        pltpu.sync_copy(ones, ones_v)
    plsc.subcore_barrier()

    def chunk(j, carry):
        row = wid * CH + j
        pltpu.sync_copy(srcr.at[row], src_idx)
        pltpu.sync_copy(dstr.at[row], dst_idx)
        pltpu.async_copy(table.at[src_idx], rows_v, sem).wait()
        pltpu.sync_copy(rows_v, acc_sh.at[dst_idx], add=True)
        if with_counts:
            pltpu.sync_copy(ones_v, cnt_sh.at[dst_idx], add=True)
        return carry

    lax.fori_loop(0, CH, chunk, 0)
    plsc.subcore_barrier()

    pltpu.sync_copy(acc_sh.at[pl.ds(s * SLAB, SLAB)],
                    acc_out.at[c, pl.ds(s * SLAB, SLAB)])
    if with_counts:
        pltpu.sync_copy(cnt_sh.at[pl.ds(s * SLAB, SLAB)],
                        cnt_out.at[c, pl.ds(s * SLAB, SLAB)])


def _make_seg_call(with_counts):
    mesh = plsc.VectorSubcoreMesh(core_axis_name="c", subcore_axis_name="s",
                                  num_cores=NC, num_subcores=NS)
    out_type = [jax.ShapeDtypeStruct((NC, NP, 16), jnp.float32)]
    if with_counts:
        out_type.append(jax.ShapeDtypeStruct((NC, NP, 16), jnp.float32))
    scratch = [
        pltpu.VMEM((K,), jnp.int32),
        pltpu.VMEM((K,), jnp.int32),
        pltpu.VMEM((K, 16), jnp.float32),
        pltpu.VMEM((K, 16), jnp.float32),
        pltpu.VMEM_SHARED((NP, 16), jnp.float32),
        pltpu.VMEM_SHARED((NP, 16), jnp.float32),
        pltpu.SemaphoreType.DMA,
    ]

    if with_counts:
        def body(table, srcr, dstr, zrows, zcnt, ones, acc_out, cnt_out,
                 *scr):
            _seg_kernel_body(True, table, srcr, dstr, zrows, zcnt, ones,
                             acc_out, cnt_out, *scr)
    else:
        def body(table, srcr, dstr, zrows, zcnt, ones, acc_out, *scr):
            _seg_kernel_body(False, table, srcr, dstr, zrows, zcnt, ones,
                             acc_out, None, *scr)

    return pl.kernel(body, out_type=tuple(out_type), mesh=mesh,
                     scratch_types=scratch,
                     compiler_params=pltpu.CompilerParams(
                         use_tc_tiling_on_sc=False))


_seg_with_cnt = _make_seg_call(True)
_seg_no_cnt = _make_seg_call(False)


def _pre_body(x_ref, wl_ref, wr_ref, t1_ref, xr1_ref):
    x = x_ref[...]
    t1_ref[...] = jnp.dot(x, wl_ref[...], preferred_element_type=jnp.float32)
    xr1_ref[...] = jnp.dot(x, wr_ref[...], preferred_element_type=jnp.float32)


def _mid_body(acc_ref, cnt_ref, xr1_ref, b1_ref, wr2_ref, b2_ref,
              h1_ref, r2_ref, cntc_ref):
    cnt = jnp.maximum(cnt_ref[0] + cnt_ref[1], 1.0)
    agg = (acc_ref[0] + acc_ref[1]) / cnt
    h1 = jnp.maximum(agg + xr1_ref[...] + b1_ref[...], 0.0)
    h1_ref[...] = h1
    r2_ref[...] = jnp.dot(h1, wr2_ref[...],
                          preferred_element_type=jnp.float32) + b2_ref[...]
    cntc_ref[...] = cnt


def _post_body(acc_ref, cnt_ref, r2_ref, wl2_ref, wf1_ref, bf1_ref,
               wf2_ref, bf2_ref, wf3_ref, bf3_ref, out_ref):
    agg2 = (acc_ref[0] + acc_ref[1]) / cnt_ref[...]
    h2 = jnp.maximum(
        jnp.dot(agg2, wl2_ref[...], preferred_element_type=jnp.float32)
        + r2_ref[...], 0.0)
    h3 = jnp.maximum(
        jnp.dot(h2, wf1_ref[...], preferred_element_type=jnp.float32)
        + bf1_ref[...], 0.0)
    h4 = jnp.maximum(
        jnp.dot(h3, wf2_ref[...], preferred_element_type=jnp.float32)
        + bf2_ref[...], 0.0)
    out_ref[...] = (jnp.dot(h4, wf3_ref[...],
                            preferred_element_type=jnp.float32)
                    + bf3_ref[...])


def kernel(x, edge_index, Wl1, Wr1, b1, Wl2, Wr2, b2,
           Wf1, bf1, Wf2, bf2, Wf3, bf3):
    src = edge_index[0]
    dst = edge_index[1]
    pad = EPAD - E
    srcr = jnp.concatenate(
        [src, jnp.zeros((pad,), jnp.int32)]).reshape(ROWS, K)
    dstr = jnp.concatenate(
        [dst, jnp.full((pad,), N, jnp.int32)]).reshape(ROWS, K)

    zrows = jnp.zeros((SLAB, 16), jnp.float32)
    zcnt = jnp.zeros((SLAB, 1), jnp.float32)
    ones = jnp.ones((K, 16), jnp.float32)

    t1, xr1 = pl.pallas_call(
        _pre_body,
        out_shape=[jax.ShapeDtypeStruct((N, 16), jnp.float32),
                   jax.ShapeDtypeStruct((N, 16), jnp.float32)],
    )(x, Wl1, Wr1)

    acc1, cnt = _seg_with_cnt(t1, srcr, dstr, zrows, zcnt, ones)

    h1, r2, cntc = pl.pallas_call(
        _mid_body,
        out_shape=[jax.ShapeDtypeStruct((N, 16), jnp.float32),
                   jax.ShapeDtypeStruct((N, 32), jnp.float32),
                   jax.ShapeDtypeStruct((N, 1), jnp.float32)],
    )(acc1[:, :N, :], cnt[:, :N, 0:1], xr1, b1.reshape(1, 16), Wr2,
      b2.reshape(1, 32))

    (acc2,) = _seg_no_cnt(h1, srcr, dstr, zrows, zcnt, ones)

    out = pl.pallas_call(
        _post_body,
        out_shape=jax.ShapeDtypeStruct((N, 64), jnp.float32),
    )(acc2[:, :N, :], cntc, r2, Wl2, Wf1, bf1.reshape(1, 64), Wf2,
      bf2.reshape(1, 128), Wf3, bf3.reshape(1, 64))
    return out


# speedup vs baseline: 12.9032x; 1.5409x over previous
import jax
import jax.numpy as jnp
from jax import lax
from jax.experimental import pallas as pl
from jax.experimental.pallas import tpu as pltpu
from jax.experimental.pallas import tpu_sc as plsc

N = 10000
E = 320000
D_IN = 128

NC = 2
NS = 16
NW = NC * NS
K = 128
NB = 8
LOOK = 4
CH = 80
EPAD = NW * K * CH
ROWS = EPAD // K
NP = 10240
SLAB = NP // NS


def _seg_kernel_body(with_counts, table, srcr, dstr, zrows, ones,
                     acc_out, cnt_out, src_idx, dst_idx, rows, ones_v,
                     acc_sh, cnt_sh, semI, semG, semS, semC):
    c = lax.axis_index("c")
    s = lax.axis_index("s")
    wid = c * NS + s
    base = wid * CH

    pltpu.sync_copy(zrows, acc_sh.at[pl.ds(s * SLAB, SLAB)])
    if with_counts:
        pltpu.sync_copy(zrows, cnt_sh.at[pl.ds(s * SLAB, SLAB)])
        pltpu.sync_copy(ones, ones_v)
    plsc.subcore_barrier()

    def group(g, carry):
        row0 = base + g * NB
        didx = []
        for t in range(NB):
            didx.append(pltpu.async_copy(srcr.at[row0 + t], src_idx[t], semI))
            didx.append(pltpu.async_copy(dstr.at[row0 + t], dst_idx[t], semI))
        for d in didx:
            d.wait()
        dg = [pltpu.async_copy(table.at[src_idx[t]], rows[t], semG)
              for t in range(NB)]
        for d in dg:
            d.wait()
        ds = [pltpu.async_copy(rows[t], acc_sh.at[dst_idx[t]], semS, add=True)
              for t in range(NB)]
        if with_counts:
            ds += [pltpu.async_copy(ones_v, cnt_sh.at[dst_idx[t]], semC,
                                    add=True)
                   for t in range(NB)]
        for d in ds:
            d.wait()
        return carry

    lax.fori_loop(0, CH // NB, group, 0)

    plsc.subcore_barrier()
    pltpu.sync_copy(acc_sh.at[pl.ds(s * SLAB, SLAB)],
                    acc_out.at[c, pl.ds(s * SLAB, SLAB)])
    if with_counts:
        pltpu.sync_copy(cnt_sh.at[pl.ds(s * SLAB, SLAB)],
                        cnt_out.at[c, pl.ds(s * SLAB, SLAB)])


def _make_seg_call(with_counts):
    mesh = plsc.VectorSubcoreMesh(core_axis_name="c", subcore_axis_name="s",
                                  num_cores=NC, num_subcores=NS)
    out_type = [jax.ShapeDtypeStruct((NC, NP, 16), jnp.float32)]
    if with_counts:
        out_type.append(jax.ShapeDtypeStruct((NC, NP, 16), jnp.float32))
    scratch = [
        [pltpu.VMEM((K,), jnp.int32) for _ in range(NB)],
        [pltpu.VMEM((K,), jnp.int32) for _ in range(NB)],
        [pltpu.VMEM((K, 16), jnp.float32) for _ in range(NB)],
        pltpu.VMEM((K, 16), jnp.float32),
        pltpu.VMEM_SHARED((NP, 16), jnp.float32),
        pltpu.VMEM_SHARED((NP, 16), jnp.float32),
        pltpu.SemaphoreType.DMA,
        pltpu.SemaphoreType.DMA,
        pltpu.SemaphoreType.DMA,
        pltpu.SemaphoreType.DMA,
    ]

    if with_counts:
        def body(table, srcr, dstr, zrows, ones, acc_out, cnt_out, *scr):
            _seg_kernel_body(True, table, srcr, dstr, zrows, ones,
                             acc_out, cnt_out, *scr)
    else:
        def body(table, srcr, dstr, zrows, ones, acc_out, *scr):
            _seg_kernel_body(False, table, srcr, dstr, zrows, ones,
                             acc_out, None, *scr)

    return pl.kernel(body, out_type=tuple(out_type), mesh=mesh,
                     scratch_types=scratch,
                     compiler_params=pltpu.CompilerParams(
                         use_tc_tiling_on_sc=False))


_seg_with_cnt = _make_seg_call(True)
_seg_no_cnt = _make_seg_call(False)


def _pre_body(x_ref, wl_ref, wr_ref, t1_ref, xr1_ref):
    x = x_ref[...]
    t1_ref[...] = jnp.dot(x, wl_ref[...], preferred_element_type=jnp.float32)
    xr1_ref[...] = jnp.dot(x, wr_ref[...], preferred_element_type=jnp.float32)


def _mid_body(acc_ref, cnt_ref, xr1_ref, b1_ref, wr2_ref, b2_ref,
              h1_ref, r2_ref, cntc_ref):
    cnt = jnp.maximum(cnt_ref[0] + cnt_ref[1], 1.0)
    agg = (acc_ref[0] + acc_ref[1]) / cnt
    h1 = jnp.maximum(agg + xr1_ref[...] + b1_ref[...], 0.0)
    h1_ref[...] = h1
    r2_ref[...] = jnp.dot(h1, wr2_ref[...],
                          preferred_element_type=jnp.float32) + b2_ref[...]
    cntc_ref[...] = cnt


def _post_body(acc_ref, cnt_ref, r2_ref, wl2_ref, wf1_ref, bf1_ref,
               wf2_ref, bf2_ref, wf3_ref, bf3_ref, out_ref):
    agg2 = (acc_ref[0] + acc_ref[1]) / cnt_ref[...]
    h2 = jnp.maximum(
        jnp.dot(agg2, wl2_ref[...], preferred_element_type=jnp.float32)
        + r2_ref[...], 0.0)
    h3 = jnp.maximum(
        jnp.dot(h2, wf1_ref[...], preferred_element_type=jnp.float32)
        + bf1_ref[...], 0.0)
    h4 = jnp.maximum(
        jnp.dot(h3, wf2_ref[...], preferred_element_type=jnp.float32)
        + bf2_ref[...], 0.0)
    out_ref[...] = (jnp.dot(h4, wf3_ref[...],
                            preferred_element_type=jnp.float32)
                    + bf3_ref[...])


def kernel(x, edge_index, Wl1, Wr1, b1, Wl2, Wr2, b2,
           Wf1, bf1, Wf2, bf2, Wf3, bf3):
    src = edge_index[0]
    dst = edge_index[1]
    pad = EPAD - E
    srcr = jnp.concatenate(
        [src, jnp.zeros((pad,), jnp.int32)]).reshape(ROWS, K)
    dstr = jnp.concatenate(
        [dst, jnp.full((pad,), N, jnp.int32)]).reshape(ROWS, K)

    zrows = jnp.zeros((SLAB, 16), jnp.float32)
    ones = jnp.ones((K, 16), jnp.float32)

    t1, xr1 = pl.pallas_call(
        _pre_body,
        out_shape=[jax.ShapeDtypeStruct((N, 16), jnp.float32),
                   jax.ShapeDtypeStruct((N, 16), jnp.float32)],
    )(x, Wl1, Wr1)

    acc1, cnt = _seg_with_cnt(t1, srcr, dstr, zrows, ones)

    h1, r2, cntc = pl.pallas_call(
        _mid_body,
        out_shape=[jax.ShapeDtypeStruct((N, 16), jnp.float32),
                   jax.ShapeDtypeStruct((N, 32), jnp.float32),
                   jax.ShapeDtypeStruct((N, 1), jnp.float32)],
    )(acc1[:, :N, :], cnt[:, :N, 0:1], xr1, b1.reshape(1, 16), Wr2,
      b2.reshape(1, 32))

    (acc2,) = _seg_no_cnt(h1, srcr, dstr, zrows, ones)

    out = pl.pallas_call(
        _post_body,
        out_shape=jax.ShapeDtypeStruct((N, 64), jnp.float32),
    )(acc2[:, :N, :], cntc, r2, Wl2, Wf1, bf1.reshape(1, 64), Wf2,
      bf2.reshape(1, 128), Wf3, bf3.reshape(1, 64))
    return out


# speedup vs baseline: 17.8563x; 1.3839x over previous
import jax
import jax.numpy as jnp
from jax import lax
from jax.experimental import pallas as pl
from jax.experimental.pallas import tpu as pltpu
from jax.experimental.pallas import tpu_sc as plsc

N = 10000
E = 320000
D_IN = 128

NC = 2
NS = 16
NW = NC * NS
K = 128
NB = 8
LOOK = 4
CH = 80
EPAD = NW * K * CH
ROWS = EPAD // K
NP = 10240
SLAB = NP // NS


def _seg_kernel_body(with_counts, table, srcr, dstr, zrows, ones,
                     acc_out, cnt_out, src_idx, dst_idx, rows, ones_v,
                     acc_sh, cnt_sh, tab_sh, semI, semG, semS, semC):
    c = lax.axis_index("c")
    s = lax.axis_index("s")
    wid = c * NS + s
    base = wid * CH

    pltpu.sync_copy(zrows, acc_sh.at[pl.ds(s * SLAB, SLAB)])
    pltpu.sync_copy(table.at[pl.ds(s * (N // NS), N // NS)],
                    tab_sh.at[pl.ds(s * (N // NS), N // NS)])
    if with_counts:
        pltpu.sync_copy(zrows, cnt_sh.at[pl.ds(s * SLAB, SLAB)])
        pltpu.sync_copy(ones, ones_v)
    plsc.subcore_barrier()

    def group(g, carry):
        row0 = base + g * NB
        didx = []
        for t in range(NB):
            didx.append(pltpu.async_copy(srcr.at[row0 + t], src_idx[t], semI))
            didx.append(pltpu.async_copy(dstr.at[row0 + t], dst_idx[t], semI))
        for d in didx:
            d.wait()
        dg = [pltpu.async_copy(tab_sh.at[src_idx[t]], rows[t], semG)
              for t in range(NB)]
        for d in dg:
            d.wait()
        ds = [pltpu.async_copy(rows[t], acc_sh.at[dst_idx[t]], semS, add=True)
              for t in range(NB)]
        if with_counts:
            ds += [pltpu.async_copy(ones_v, cnt_sh.at[dst_idx[t]], semC,
                                    add=True)
                   for t in range(NB)]
        for d in ds:
            d.wait()
        return carry

    lax.fori_loop(0, CH // NB, group, 0)

    plsc.subcore_barrier()
    pltpu.sync_copy(acc_sh.at[pl.ds(s * SLAB, SLAB)],
                    acc_out.at[c, pl.ds(s * SLAB, SLAB)])
    if with_counts:
        pltpu.sync_copy(cnt_sh.at[pl.ds(s * SLAB, SLAB)],
                        cnt_out.at[c, pl.ds(s * SLAB, SLAB)])


def _make_seg_call(with_counts):
    mesh = plsc.VectorSubcoreMesh(core_axis_name="c", subcore_axis_name="s",
                                  num_cores=NC, num_subcores=NS)
    out_type = [jax.ShapeDtypeStruct((NC, NP, 16), jnp.float32)]
    if with_counts:
        out_type.append(jax.ShapeDtypeStruct((NC, NP, 16), jnp.float32))
    scratch = [
        [pltpu.VMEM((K,), jnp.int32) for _ in range(NB)],
        [pltpu.VMEM((K,), jnp.int32) for _ in range(NB)],
        [pltpu.VMEM((K, 16), jnp.float32) for _ in range(NB)],
        pltpu.VMEM((K, 16), jnp.float32),
        pltpu.VMEM_SHARED((NP, 16), jnp.float32),
        pltpu.VMEM_SHARED((NP, 16), jnp.float32),
        pltpu.VMEM_SHARED((N, 16), jnp.float32),
        pltpu.SemaphoreType.DMA,
        pltpu.SemaphoreType.DMA,
        pltpu.SemaphoreType.DMA,
        pltpu.SemaphoreType.DMA,
    ]

    if with_counts:
        def body(table, srcr, dstr, zrows, ones, acc_out, cnt_out, *scr):
            _seg_kernel_body(True, table, srcr, dstr, zrows, ones,
                             acc_out, cnt_out, *scr)
    else:
        def body(table, srcr, dstr, zrows, ones, acc_out, *scr):
            _seg_kernel_body(False, table, srcr, dstr, zrows, ones,
                             acc_out, None, *scr)

    return pl.kernel(body, out_type=tuple(out_type), mesh=mesh,
                     scratch_types=scratch,
                     compiler_params=pltpu.CompilerParams(
                         use_tc_tiling_on_sc=False))


_seg_with_cnt = _make_seg_call(True)
_seg_no_cnt = _make_seg_call(False)


def _pre_body(x_ref, wl_ref, wr_ref, t1_ref, xr1_ref):
    x = x_ref[...]
    t1_ref[...] = jnp.dot(x, wl_ref[...], preferred_element_type=jnp.float32)
    xr1_ref[...] = jnp.dot(x, wr_ref[...], preferred_element_type=jnp.float32)


def _mid_body(acc_ref, cnt_ref, xr1_ref, b1_ref, wr2_ref, b2_ref,
              h1_ref, r2_ref, cntc_ref):
    cnt = jnp.maximum(cnt_ref[0] + cnt_ref[1], 1.0)
    agg = (acc_ref[0] + acc_ref[1]) / cnt
    h1 = jnp.maximum(agg + xr1_ref[...] + b1_ref[...], 0.0)
    h1_ref[...] = h1
    r2_ref[...] = jnp.dot(h1, wr2_ref[...],
                          preferred_element_type=jnp.float32) + b2_ref[...]
    cntc_ref[...] = cnt


def _post_body(acc_ref, cnt_ref, r2_ref, wl2_ref, wf1_ref, bf1_ref,
               wf2_ref, bf2_ref, wf3_ref, bf3_ref, out_ref):
    agg2 = (acc_ref[0] + acc_ref[1]) / cnt_ref[...]
    h2 = jnp.maximum(
        jnp.dot(agg2, wl2_ref[...], preferred_element_type=jnp.float32)
        + r2_ref[...], 0.0)
    h3 = jnp.maximum(
        jnp.dot(h2, wf1_ref[...], preferred_element_type=jnp.float32)
        + bf1_ref[...], 0.0)
    h4 = jnp.maximum(
        jnp.dot(h3, wf2_ref[...], preferred_element_type=jnp.float32)
        + bf2_ref[...], 0.0)
    out_ref[...] = (jnp.dot(h4, wf3_ref[...],
                            preferred_element_type=jnp.float32)
                    + bf3_ref[...])


def kernel(x, edge_index, Wl1, Wr1, b1, Wl2, Wr2, b2,
           Wf1, bf1, Wf2, bf2, Wf3, bf3):
    src = edge_index[0]
    dst = edge_index[1]
    pad = EPAD - E
    srcr = jnp.concatenate(
        [src, jnp.zeros((pad,), jnp.int32)]).reshape(ROWS, K)
    dstr = jnp.concatenate(
        [dst, jnp.full((pad,), N, jnp.int32)]).reshape(ROWS, K)

    zrows = jnp.zeros((SLAB, 16), jnp.float32)
    ones = jnp.ones((K, 16), jnp.float32)

    t1, xr1 = pl.pallas_call(
        _pre_body,
        out_shape=[jax.ShapeDtypeStruct((N, 16), jnp.float32),
                   jax.ShapeDtypeStruct((N, 16), jnp.float32)],
    )(x, Wl1, Wr1)

    acc1, cnt = _seg_with_cnt(t1, srcr, dstr, zrows, ones)

    h1, r2, cntc = pl.pallas_call(
        _mid_body,
        out_shape=[jax.ShapeDtypeStruct((N, 16), jnp.float32),
                   jax.ShapeDtypeStruct((N, 32), jnp.float32),
                   jax.ShapeDtypeStruct((N, 1), jnp.float32)],
    )(acc1[:, :N, :], cnt[:, :N, 0:1], xr1, b1.reshape(1, 16), Wr2,
      b2.reshape(1, 32))

    (acc2,) = _seg_no_cnt(h1, srcr, dstr, zrows, ones)

    out = pl.pallas_call(
        _post_body,
        out_shape=jax.ShapeDtypeStruct((N, 64), jnp.float32),
    )(acc2[:, :N, :], cntc, r2, Wl2, Wf1, bf1.reshape(1, 64), Wf2,
      bf2.reshape(1, 128), Wf3, bf3.reshape(1, 64))
    return out


# speedup vs baseline: 21.3881x; 1.1978x over previous
import jax
import jax.numpy as jnp
from jax import lax
from jax.experimental import pallas as pl
from jax.experimental.pallas import tpu as pltpu
from jax.experimental.pallas import tpu_sc as plsc

N = 10000
E = 320000
D_IN = 128

NC = 2
NS = 16
NW = NC * NS
K = 128
NB = 8
CH = 80
EPAD = NW * K * CH
ROWS = EPAD // K
NP = 10240
SLAB = NP // NS
NSL = N // NS


def _edge_loop(wid, srcr, dstr, src_idx, dst_idx, rows, tab_sh, acc_sh,
               cnt_sh, ones_v, semI, semG, semS, semC, with_counts):
    base = wid * CH

    def group(g, carry):
        row0 = base + g * NB
        didx = []
        for t in range(NB):
            didx.append(pltpu.async_copy(srcr.at[row0 + t], src_idx[t], semI))
            didx.append(pltpu.async_copy(dstr.at[row0 + t], dst_idx[t], semI))
        for d in didx:
            d.wait()
        dg = [pltpu.async_copy(tab_sh.at[src_idx[t]], rows[t], semG)
              for t in range(NB)]
        for d in dg:
            d.wait()
        ds = [pltpu.async_copy(rows[t], acc_sh.at[dst_idx[t]], semS, add=True)
              for t in range(NB)]
        if with_counts:
            ds += [pltpu.async_copy(ones_v, cnt_sh.at[dst_idx[t]], semC,
                                    add=True)
                   for t in range(NB)]
        for d in ds:
            d.wait()
        return carry

    lax.fori_loop(0, CH // NB, group, 0)


def _seg1_body(table, srcr, dstr, zrows, ones, acc_out, cnt_out,
               src_idx, dst_idx, rows, ones_v, acc_sh, cnt_sh, tab_sh,
               semI, semG, semS, semC):
    c = lax.axis_index("c")
    s = lax.axis_index("s")
    wid = c * NS + s

    pltpu.sync_copy(zrows, acc_sh.at[pl.ds(s * SLAB, SLAB)])
    pltpu.sync_copy(zrows, cnt_sh.at[pl.ds(s * SLAB, SLAB)])
    pltpu.sync_copy(ones, ones_v)
    pltpu.sync_copy(table.at[pl.ds(s * NSL, NSL)],
                    tab_sh.at[pl.ds(s * NSL, NSL)])
    plsc.subcore_barrier()

    _edge_loop(wid, srcr, dstr, src_idx, dst_idx, rows, tab_sh, acc_sh,
               cnt_sh, ones_v, semI, semG, semS, semC, True)

    plsc.subcore_barrier()
    pltpu.sync_copy(acc_sh.at[pl.ds(s * SLAB, SLAB)],
                    acc_out.at[c, pl.ds(s * SLAB, SLAB)])
    pltpu.sync_copy(cnt_sh.at[pl.ds(s * SLAB, SLAB)],
                    cnt_out.at[c, pl.ds(s * SLAB, SLAB)])


def _seg2_body(acc1, cnt1, xr1, srcr, dstr, zrows,
               acc_out, h1_out, cntc_out,
               src_idx, dst_idx, rows, a0b, a1b, c0b, c1b, xb, hb, cnb,
               acc_sh, tab_sh, semI, semG, semS):
    c = lax.axis_index("c")
    s = lax.axis_index("s")
    wid = c * NS + s
    r0 = s * NSL

    pltpu.sync_copy(zrows, acc_sh.at[pl.ds(s * SLAB, SLAB)])
    pltpu.sync_copy(acc1.at[0, pl.ds(r0, NSL)], a0b)
    pltpu.sync_copy(acc1.at[1, pl.ds(r0, NSL)], a1b)
    pltpu.sync_copy(cnt1.at[0, pl.ds(r0, NSL)], c0b)
    pltpu.sync_copy(cnt1.at[1, pl.ds(r0, NSL)], c1b)
    pltpu.sync_copy(xr1.at[pl.ds(r0, NSL)], xb)

    def stage(r, carry):
        cn = jnp.maximum(c0b[r] + c1b[r], 1.0)
        hb[r] = jnp.maximum((a0b[r] + a1b[r]) / cn + xb[r], 0.0)
        cnb[r] = cn
        return carry

    lax.fori_loop(0, NSL, stage, 0)
    pltpu.sync_copy(hb, tab_sh.at[pl.ds(r0, NSL)])

    @pl.when(c == 0)
    def _():
        pltpu.sync_copy(hb, h1_out.at[pl.ds(r0, NSL)])
        pltpu.sync_copy(cnb, cntc_out.at[pl.ds(r0, NSL)])

    plsc.subcore_barrier()

    _edge_loop(wid, srcr, dstr, src_idx, dst_idx, rows, tab_sh, acc_sh,
               None, None, semI, semG, semS, None, False)

    plsc.subcore_barrier()
    pltpu.sync_copy(acc_sh.at[pl.ds(s * SLAB, SLAB)],
                    acc_out.at[c, pl.ds(s * SLAB, SLAB)])


_MESH = plsc.VectorSubcoreMesh(core_axis_name="c", subcore_axis_name="s",
                               num_cores=NC, num_subcores=NS)
_SC_PARAMS = pltpu.CompilerParams(use_tc_tiling_on_sc=False)

_seg1 = pl.kernel(
    _seg1_body,
    out_type=(jax.ShapeDtypeStruct((NC, NP, 16), jnp.float32),
              jax.ShapeDtypeStruct((NC, NP, 16), jnp.float32)),
    mesh=_MESH,
    scratch_types=[
        [pltpu.VMEM((K,), jnp.int32) for _ in range(NB)],
        [pltpu.VMEM((K,), jnp.int32) for _ in range(NB)],
        [pltpu.VMEM((K, 16), jnp.float32) for _ in range(NB)],
        pltpu.VMEM((K, 16), jnp.float32),
        pltpu.VMEM_SHARED((NP, 16), jnp.float32),
        pltpu.VMEM_SHARED((NP, 16), jnp.float32),
        pltpu.VMEM_SHARED((N, 16), jnp.float32),
        pltpu.SemaphoreType.DMA,
        pltpu.SemaphoreType.DMA,
        pltpu.SemaphoreType.DMA,
        pltpu.SemaphoreType.DMA,
    ],
    compiler_params=_SC_PARAMS)

_seg2 = pl.kernel(
    _seg2_body,
    out_type=(jax.ShapeDtypeStruct((NC, NP, 16), jnp.float32),
              jax.ShapeDtypeStruct((N, 16), jnp.float32),
              jax.ShapeDtypeStruct((N, 16), jnp.float32)),
    mesh=_MESH,
    scratch_types=[
        [pltpu.VMEM((K,), jnp.int32) for _ in range(NB)],
        [pltpu.VMEM((K,), jnp.int32) for _ in range(NB)],
        [pltpu.VMEM((K, 16), jnp.float32) for _ in range(NB)],
        pltpu.VMEM((NSL, 16), jnp.float32),
        pltpu.VMEM((NSL, 16), jnp.float32),
        pltpu.VMEM((NSL, 16), jnp.float32),
        pltpu.VMEM((NSL, 16), jnp.float32),
        pltpu.VMEM((NSL, 16), jnp.float32),
        pltpu.VMEM((NSL, 16), jnp.float32),
        pltpu.VMEM((NSL, 16), jnp.float32),
        pltpu.VMEM_SHARED((NP, 16), jnp.float32),
        pltpu.VMEM_SHARED((N, 16), jnp.float32),
        pltpu.SemaphoreType.DMA,
        pltpu.SemaphoreType.DMA,
        pltpu.SemaphoreType.DMA,
    ],
    compiler_params=_SC_PARAMS)


def _pre_body(x_ref, wl_ref, wr_ref, b1_ref, t1_ref, xr1_ref):
    x = x_ref[...]
    t1_ref[...] = jnp.dot(x, wl_ref[...], preferred_element_type=jnp.float32)
    xr1_ref[...] = (jnp.dot(x, wr_ref[...], preferred_element_type=jnp.float32)
                    + b1_ref[...])


def _post_body(acc_ref, cnt_ref, h1_ref, wr2_ref, b2_ref, wl2_ref,
               wf1_ref, bf1_ref, wf2_ref, bf2_ref, wf3_ref, bf3_ref,
               out_ref):
    agg2 = (acc_ref[0, :N, :] + acc_ref[1, :N, :]) / cnt_ref[...][:, 0:1]
    r2 = jnp.dot(h1_ref[...], wr2_ref[...],
                 preferred_element_type=jnp.float32) + b2_ref[...]
    h2 = jnp.maximum(
        jnp.dot(agg2, wl2_ref[...], preferred_element_type=jnp.float32)
        + r2, 0.0)
    h3 = jnp.maximum(
        jnp.dot(h2, wf1_ref[...], preferred_element_type=jnp.float32)
        + bf1_ref[...], 0.0)
    h4 = jnp.maximum(
        jnp.dot(h3, wf2_ref[...], preferred_element_type=jnp.float32)
        + bf2_ref[...], 0.0)
    out_ref[...] = (jnp.dot(h4, wf3_ref[...],
                            preferred_element_type=jnp.float32)
                    + bf3_ref[...])


def kernel(x, edge_index, Wl1, Wr1, b1, Wl2, Wr2, b2,
           Wf1, bf1, Wf2, bf2, Wf3, bf3):
    src = edge_index[0]
    dst = edge_index[1]
    pad = EPAD - E
    srcr = jnp.concatenate(
        [src, jnp.zeros((pad,), jnp.int32)]).reshape(ROWS, K)
    dstr = jnp.concatenate(
        [dst, jnp.full((pad,), N, jnp.int32)]).reshape(ROWS, K)

    zrows = jnp.zeros((SLAB, 16), jnp.float32)
    ones = jnp.ones((K, 16), jnp.float32)

    t1, xr1 = pl.pallas_call(
        _pre_body,
        out_shape=[jax.ShapeDtypeStruct((N, 16), jnp.float32),
                   jax.ShapeDtypeStruct((N, 16), jnp.float32)],
    )(x, Wl1, Wr1, b1.reshape(1, 16))

    acc1, cnt1 = _seg1(t1, srcr, dstr, zrows, ones)
    acc2, h1, cntc = _seg2(acc1, cnt1, xr1, srcr, dstr, zrows)

    out = pl.pallas_call(
        _post_body,
        out_shape=jax.ShapeDtypeStruct((N, 64), jnp.float32),
    )(acc2, cntc, h1, Wr2, b2.reshape(1, 32), Wl2, Wf1, bf1.reshape(1, 64),
      Wf2, bf2.reshape(1, 128), Wf3, bf3.reshape(1, 64))
    return out


# speedup vs baseline: 21.6850x; 1.0139x over previous
import jax
import jax.numpy as jnp
from jax import lax
from jax.experimental import pallas as pl
from jax.experimental.pallas import tpu as pltpu
from jax.experimental.pallas import tpu_sc as plsc

N = 10000
E = 320000
D_IN = 128

NC = 2
NS = 16
NW = NC * NS
K = 128
NB = 8
CH = 80
EPAD = NW * K * CH
ROWS = EPAD // K
NP = 10240
SLAB = NP // NS
NSL = N // NS


def _edge_loop(wid, srcr, dstr, src_idx, dst_idx, rows, tab_sh, acc_sh,
               cnt_sh, ones_v, semI, semG, semS, semC, with_counts):
    base = wid * CH

    def group(g, carry):
        row0 = base + g * NB
        didx = []
        for t in range(NB):
            didx.append(pltpu.async_copy(srcr.at[row0 + t], src_idx[t], semI))
            didx.append(pltpu.async_copy(dstr.at[row0 + t], dst_idx[t], semI))
        for d in didx:
            d.wait()
        dg = [pltpu.async_copy(tab_sh.at[src_idx[t]], rows[t], semG)
              for t in range(NB)]
        for d in dg:
            d.wait()
        ds = [pltpu.async_copy(rows[t], acc_sh.at[dst_idx[t]], semS, add=True)
              for t in range(NB)]
        if with_counts:
            ds += [pltpu.async_copy(ones_v, cnt_sh.at[dst_idx[t]], semC,
                                    add=True)
                   for t in range(NB)]
        for d in ds:
            d.wait()
        return carry

    lax.fori_loop(0, CH // NB, group, 0)


def _seg1_body(table, srcr, dstr, zrows, ones, acc_out, cnt_out,
               src_idx, dst_idx, rows, ones_v, acc_sh, cnt_sh, tab_sh,
               semI, semG, semS, semC):
    c = lax.axis_index("c")
    s = lax.axis_index("s")
    wid = c * NS + s

    pltpu.sync_copy(zrows, acc_sh.at[pl.ds(s * SLAB, SLAB)])
    pltpu.sync_copy(zrows, cnt_sh.at[pl.ds(s * SLAB, SLAB)])
    pltpu.sync_copy(ones, ones_v)
    pltpu.sync_copy(table.at[pl.ds(s * NSL, NSL)],
                    tab_sh.at[pl.ds(s * NSL, NSL)])
    plsc.subcore_barrier()

    _edge_loop(wid, srcr, dstr, src_idx, dst_idx, rows, tab_sh, acc_sh,
               cnt_sh, ones_v, semI, semG, semS, semC, True)

    plsc.subcore_barrier()
    pltpu.sync_copy(acc_sh.at[pl.ds(s * SLAB, SLAB)],
                    acc_out.at[c, pl.ds(s * SLAB, SLAB)])
    pltpu.sync_copy(cnt_sh.at[pl.ds(s * SLAB, SLAB)],
                    cnt_out.at[c, pl.ds(s * SLAB, SLAB)])


def _seg2_body(acc1, cnt1, xr1, srcr, dstr, zrows,
               cat_out,
               src_idx, dst_idx, rows, a0b, a1b, c0b, c1b, xb, hb, cnb,
               acc_sh, tab_sh, semI, semG, semS):
    c = lax.axis_index("c")
    s = lax.axis_index("s")
    wid = c * NS + s
    r0 = s * NSL

    pltpu.sync_copy(zrows, acc_sh.at[pl.ds(s * SLAB, SLAB)])
    pltpu.sync_copy(acc1.at[0, pl.ds(r0, NSL)], a0b)
    pltpu.sync_copy(acc1.at[1, pl.ds(r0, NSL)], a1b)
    pltpu.sync_copy(cnt1.at[0, pl.ds(r0, NSL)], c0b)
    pltpu.sync_copy(cnt1.at[1, pl.ds(r0, NSL)], c1b)
    pltpu.sync_copy(xr1.at[pl.ds(r0, NSL)], xb)

    def stage(r, carry):
        cn = jnp.maximum(c0b[r] + c1b[r], 1.0)
        hb[r] = jnp.maximum((a0b[r] + a1b[r]) / cn + xb[r], 0.0)
        cnb[r] = cn
        return carry

    lax.fori_loop(0, NSL, stage, 0)
    pltpu.sync_copy(hb, tab_sh.at[pl.ds(r0, NSL)])

    @pl.when(c == 0)
    def _():
        pltpu.sync_copy(hb, cat_out.at[pl.ds(2 * NP + r0, NSL)])
        pltpu.sync_copy(cnb, cat_out.at[pl.ds(2 * NP + N + r0, NSL)])

    plsc.subcore_barrier()

    _edge_loop(wid, srcr, dstr, src_idx, dst_idx, rows, tab_sh, acc_sh,
               None, None, semI, semG, semS, None, False)

    plsc.subcore_barrier()
    pltpu.sync_copy(acc_sh.at[pl.ds(s * SLAB, SLAB)],
                    cat_out.at[pl.ds(c * NP + s * SLAB, SLAB)])


_MESH = plsc.VectorSubcoreMesh(core_axis_name="c", subcore_axis_name="s",
                               num_cores=NC, num_subcores=NS)
_SC_PARAMS = pltpu.CompilerParams(use_tc_tiling_on_sc=False)

_seg1 = pl.kernel(
    _seg1_body,
    out_type=(jax.ShapeDtypeStruct((NC, NP, 16), jnp.float32),
              jax.ShapeDtypeStruct((NC, NP, 16), jnp.float32)),
    mesh=_MESH,
    scratch_types=[
        [pltpu.VMEM((K,), jnp.int32) for _ in range(NB)],
        [pltpu.VMEM((K,), jnp.int32) for _ in range(NB)],
        [pltpu.VMEM((K, 16), jnp.float32) for _ in range(NB)],
        pltpu.VMEM((K, 16), jnp.float32),
        pltpu.VMEM_SHARED((NP, 16), jnp.float32),
        pltpu.VMEM_SHARED((NP, 16), jnp.float32),
        pltpu.VMEM_SHARED((N, 16), jnp.float32),
        pltpu.SemaphoreType.DMA,
        pltpu.SemaphoreType.DMA,
        pltpu.SemaphoreType.DMA,
        pltpu.SemaphoreType.DMA,
    ],
    compiler_params=_SC_PARAMS)

_seg2 = pl.kernel(
    _seg2_body,
    out_type=jax.ShapeDtypeStruct((2 * NP + 2 * N, 16), jnp.float32),
    mesh=_MESH,
    scratch_types=[
        [pltpu.VMEM((K,), jnp.int32) for _ in range(NB)],
        [pltpu.VMEM((K,), jnp.int32) for _ in range(NB)],
        [pltpu.VMEM((K, 16), jnp.float32) for _ in range(NB)],
        pltpu.VMEM((NSL, 16), jnp.float32),
        pltpu.VMEM((NSL, 16), jnp.float32),
        pltpu.VMEM((NSL, 16), jnp.float32),
        pltpu.VMEM((NSL, 16), jnp.float32),
        pltpu.VMEM((NSL, 16), jnp.float32),
        pltpu.VMEM((NSL, 16), jnp.float32),
        pltpu.VMEM((NSL, 16), jnp.float32),
        pltpu.VMEM_SHARED((NP, 16), jnp.float32),
        pltpu.VMEM_SHARED((N, 16), jnp.float32),
        pltpu.SemaphoreType.DMA,
        pltpu.SemaphoreType.DMA,
        pltpu.SemaphoreType.DMA,
    ],
    compiler_params=_SC_PARAMS)


def _pre_a_body(x_ref, wl_ref, t1_ref):
    t1_ref[...] = jnp.dot(x_ref[...], wl_ref[...],
                          preferred_element_type=jnp.float32)


def _pre_b_body(x_ref, wr_ref, b1_ref, xr1_ref):
    xr1_ref[...] = (jnp.dot(x_ref[...], wr_ref[...],
                            preferred_element_type=jnp.float32)
                    + b1_ref[...])


def _post_body(cat_ref, wr2_ref, b2_ref, wl2_ref,
               wf1_ref, bf1_ref, wf2_ref, bf2_ref, wf3_ref, bf3_ref,
               out_ref):
    agg2 = ((cat_ref[pl.ds(0, N), :] + cat_ref[pl.ds(NP, N), :])
            / cat_ref[pl.ds(2 * NP + N, N), :][:, 0:1])
    r2 = jnp.dot(cat_ref[pl.ds(2 * NP, N), :], wr2_ref[...],
                 preferred_element_type=jnp.float32) + b2_ref[...]
    h2 = jnp.maximum(
        jnp.dot(agg2, wl2_ref[...], preferred_element_type=jnp.float32)
        + r2, 0.0)
    h3 = jnp.maximum(
        jnp.dot(h2, wf1_ref[...], preferred_element_type=jnp.float32)
        + bf1_ref[...], 0.0)
    h4 = jnp.maximum(
        jnp.dot(h3, wf2_ref[...], preferred_element_type=jnp.float32)
        + bf2_ref[...], 0.0)
    out_ref[...] = (jnp.dot(h4, wf3_ref[...],
                            preferred_element_type=jnp.float32)
                    + bf3_ref[...])


def kernel(x, edge_index, Wl1, Wr1, b1, Wl2, Wr2, b2,
           Wf1, bf1, Wf2, bf2, Wf3, bf3):
    src = edge_index[0]
    dst = edge_index[1]
    pad = EPAD - E
    srcr = jnp.concatenate(
        [src, jnp.zeros((pad,), jnp.int32)]).reshape(ROWS, K)
    dstr = jnp.concatenate(
        [dst, jnp.full((pad,), N, jnp.int32)]).reshape(ROWS, K)

    zrows = jnp.zeros((SLAB, 16), jnp.float32)
    ones = jnp.ones((K, 16), jnp.float32)

    t1 = pl.pallas_call(
        _pre_a_body,
        out_shape=jax.ShapeDtypeStruct((N, 16), jnp.float32),
    )(x, Wl1)
    xr1 = pl.pallas_call(
        _pre_b_body,
        out_shape=jax.ShapeDtypeStruct((N, 16), jnp.float32),
    )(x, Wr1, b1.reshape(1, 16))

    acc1, cnt1 = _seg1(t1, srcr, dstr, zrows, ones)
    cat = _seg2(acc1, cnt1, xr1, srcr, dstr, zrows)

    out = pl.pallas_call(
        _post_body,
        out_shape=jax.ShapeDtypeStruct((N, 64), jnp.float32),
    )(cat, Wr2, b2.reshape(1, 32), Wl2, Wf1, bf1.reshape(1, 64),
      Wf2, bf2.reshape(1, 128), Wf3, bf3.reshape(1, 64))
    return out


# speedup vs baseline: 23.1519x; 1.0676x over previous
import jax
import jax.numpy as jnp
from jax import lax
from jax.experimental import pallas as pl
from jax.experimental.pallas import tpu as pltpu
from jax.experimental.pallas import tpu_sc as plsc

N = 10000
E = 320000
D_IN = 128

NC = 2
NS = 16
NW = NC * NS
K = 128
NB = 8
CH = 80
EPAD = NW * K * CH
ROWS = EPAD // K
NP = 10240
SLAB = NP // NS
NSL = N // NS


def _edge_loop(wid, srcr, dstr, src_idx, dst_idx, rows, tab_sh, acc_sh,
               cnt_sh, ones_v, semI, semG, semS, semC, with_counts):
    base = wid * CH

    def group(g, carry):
        row0 = base + g * NB
        didx = []
        for t in range(NB):
            didx.append(pltpu.async_copy(srcr.at[row0 + t], src_idx[t], semI))
            didx.append(pltpu.async_copy(dstr.at[row0 + t], dst_idx[t], semI))
        for d in didx:
            d.wait()
        dg = [pltpu.async_copy(tab_sh.at[src_idx[t]], rows[t], semG)
              for t in range(NB)]
        for d in dg:
            d.wait()
        ds = [pltpu.async_copy(rows[t], acc_sh.at[dst_idx[t]], semS, add=True)
              for t in range(NB)]
        if with_counts:
            ds += [pltpu.async_copy(ones_v, cnt_sh.at[dst_idx[t]], semC,
                                    add=True)
                   for t in range(NB)]
        for d in ds:
            d.wait()
        return carry

    lax.fori_loop(0, CH // NB, group, 0)


def _seg1_body(table, srcr, dstr, zrows, ones, acc_out, cnt_out,
               src_idx, dst_idx, rows, ones_v, acc_sh, cnt_sh, tab_sh,
               semI, semG, semS, semC):
    c = lax.axis_index("c")
    s = lax.axis_index("s")
    wid = c * NS + s

    pltpu.sync_copy(zrows, acc_sh.at[pl.ds(s * SLAB, SLAB)])
    pltpu.sync_copy(zrows, cnt_sh.at[pl.ds(s * SLAB, SLAB)])
    pltpu.sync_copy(ones, ones_v)
    pltpu.sync_copy(table.at[pl.ds(s * NSL, NSL)],
                    tab_sh.at[pl.ds(s * NSL, NSL)])
    plsc.subcore_barrier()

    _edge_loop(wid, srcr, dstr, src_idx, dst_idx, rows, tab_sh, acc_sh,
               cnt_sh, ones_v, semI, semG, semS, semC, True)

    plsc.subcore_barrier()
    pltpu.sync_copy(acc_sh.at[pl.ds(s * SLAB, SLAB)],
                    acc_out.at[c, pl.ds(s * SLAB, SLAB)])
    pltpu.sync_copy(cnt_sh.at[pl.ds(s * SLAB, SLAB)],
                    cnt_out.at[c, pl.ds(s * SLAB, SLAB)])


def _seg2_body(acc1, cnt1, xr1, srcr, dstr, zrows,
               cat_out,
               src_idx, dst_idx, rows, a0b, a1b, c0b, c1b, xb, hb, cnb,
               acc_sh, tab_sh, semI, semG, semS):
    c = lax.axis_index("c")
    s = lax.axis_index("s")
    wid = c * NS + s
    r0 = s * NSL

    pltpu.sync_copy(zrows, acc_sh.at[pl.ds(s * SLAB, SLAB)])
    pltpu.sync_copy(acc1.at[0, pl.ds(r0, NSL)], a0b)
    pltpu.sync_copy(acc1.at[1, pl.ds(r0, NSL)], a1b)
    pltpu.sync_copy(cnt1.at[0, pl.ds(r0, NSL)], c0b)
    pltpu.sync_copy(cnt1.at[1, pl.ds(r0, NSL)], c1b)
    pltpu.sync_copy(xr1.at[pl.ds(r0, NSL)], xb)

    def stage(r, carry):
        cn = jnp.maximum(c0b[r] + c1b[r], 1.0)
        hb[r] = jnp.maximum((a0b[r] + a1b[r]) / cn + xb[r], 0.0)
        cnb[r] = cn
        return carry

    lax.fori_loop(0, NSL, stage, 0)
    pltpu.sync_copy(hb, tab_sh.at[pl.ds(r0, NSL)])

    @pl.when(c == 0)
    def _():
        pltpu.sync_copy(hb, cat_out.at[pl.ds(2 * NP + r0, NSL)])
        pltpu.sync_copy(cnb, cat_out.at[pl.ds(2 * NP + N + r0, NSL)])

    plsc.subcore_barrier()

    _edge_loop(wid, srcr, dstr, src_idx, dst_idx, rows, tab_sh, acc_sh,
               None, None, semI, semG, semS, None, False)

    plsc.subcore_barrier()
    pltpu.sync_copy(acc_sh.at[pl.ds(s * SLAB, SLAB)],
                    cat_out.at[pl.ds(c * NP + s * SLAB, SLAB)])


_MESH = plsc.VectorSubcoreMesh(core_axis_name="c", subcore_axis_name="s",
                               num_cores=NC, num_subcores=NS)
_SC_PARAMS = pltpu.CompilerParams(use_tc_tiling_on_sc=False)

_seg1 = pl.kernel(
    _seg1_body,
    out_type=(jax.ShapeDtypeStruct((NC, NP, 16), jnp.float32),
              jax.ShapeDtypeStruct((NC, NP, 16), jnp.float32)),
    mesh=_MESH,
    scratch_types=[
        [pltpu.VMEM((K,), jnp.int32) for _ in range(NB)],
        [pltpu.VMEM((K,), jnp.int32) for _ in range(NB)],
        [pltpu.VMEM((K, 16), jnp.float32) for _ in range(NB)],
        pltpu.VMEM((K, 16), jnp.float32),
        pltpu.VMEM_SHARED((NP, 16), jnp.float32),
        pltpu.VMEM_SHARED((NP, 16), jnp.float32),
        pltpu.VMEM_SHARED((N, 16), jnp.float32),
        pltpu.SemaphoreType.DMA,
        pltpu.SemaphoreType.DMA,
        pltpu.SemaphoreType.DMA,
        pltpu.SemaphoreType.DMA,
    ],
    compiler_params=_SC_PARAMS)

_seg2 = pl.kernel(
    _seg2_body,
    out_type=jax.ShapeDtypeStruct((2 * NP + 2 * N, 16), jnp.float32),
    mesh=_MESH,
    scratch_types=[
        [pltpu.VMEM((K,), jnp.int32) for _ in range(NB)],
        [pltpu.VMEM((K,), jnp.int32) for _ in range(NB)],
        [pltpu.VMEM((K, 16), jnp.float32) for _ in range(NB)],
        pltpu.VMEM((NSL, 16), jnp.float32),
        pltpu.VMEM((NSL, 16), jnp.float32),
        pltpu.VMEM((NSL, 16), jnp.float32),
        pltpu.VMEM((NSL, 16), jnp.float32),
        pltpu.VMEM((NSL, 16), jnp.float32),
        pltpu.VMEM((NSL, 16), jnp.float32),
        pltpu.VMEM((NSL, 16), jnp.float32),
        pltpu.VMEM_SHARED((NP, 16), jnp.float32),
        pltpu.VMEM_SHARED((N, 16), jnp.float32),
        pltpu.SemaphoreType.DMA,
        pltpu.SemaphoreType.DMA,
        pltpu.SemaphoreType.DMA,
    ],
    compiler_params=_SC_PARAMS)


def _pre_a_body(x_ref, wl_ref, t1_ref):
    t1_ref[...] = jnp.dot(x_ref[...], wl_ref[...],
                          preferred_element_type=jnp.float32)


def _pre_b_body(x_ref, wr_ref, b1_ref, xr1_ref):
    xr1_ref[...] = (jnp.dot(x_ref[...], wr_ref[...],
                            preferred_element_type=jnp.float32)
                    + b1_ref[...])


def _post_body(cat_ref, wr2_ref, b2_ref, wl2_ref,
               wf1_ref, bf1_ref, wf2_ref, bf2_ref, wf3_ref, bf3_ref,
               out_ref):
    a0 = cat_ref[pl.ds(0, N // 8), :]
    a1 = cat_ref[pl.ds(NP // 8, N // 8), :]
    h1 = cat_ref[pl.ds(2 * NP // 8, N // 8), :]
    cn = cat_ref[pl.ds((2 * NP + N) // 8, N // 8), :]
    agg2 = (a0 + a1) / cn
    r2 = jnp.dot(h1, wr2_ref[...],
                 preferred_element_type=jnp.float32) + b2_ref[...]
    h2 = jnp.maximum(
        jnp.dot(agg2, wl2_ref[...], preferred_element_type=jnp.float32)
        + r2, 0.0)
    h3 = jnp.maximum(
        jnp.dot(h2, wf1_ref[...], preferred_element_type=jnp.float32)
        + bf1_ref[...], 0.0)
    h4 = jnp.maximum(
        jnp.dot(h3, wf2_ref[...], preferred_element_type=jnp.float32)
        + bf2_ref[...], 0.0)
    out_ref[...] = (jnp.dot(h4, wf3_ref[...],
                            preferred_element_type=jnp.float32)
                    + bf3_ref[...])


def kernel(x, edge_index, Wl1, Wr1, b1, Wl2, Wr2, b2,
           Wf1, bf1, Wf2, bf2, Wf3, bf3):
    src = edge_index[0]
    dst = edge_index[1]
    pad = EPAD - E
    srcr = jnp.concatenate(
        [src, jnp.zeros((pad,), jnp.int32)]).reshape(ROWS, K)
    dstr = jnp.concatenate(
        [dst, jnp.full((pad,), N, jnp.int32)]).reshape(ROWS, K)

    zrows = jnp.zeros((SLAB, 16), jnp.float32)
    ones = jnp.ones((K, 16), jnp.float32)

    t1 = pl.pallas_call(
        _pre_a_body,
        out_shape=jax.ShapeDtypeStruct((N, 16), jnp.float32),
    )(x, Wl1)
    xr1 = pl.pallas_call(
        _pre_b_body,
        out_shape=jax.ShapeDtypeStruct((N, 16), jnp.float32),
    )(x, Wr1, b1.reshape(1, 16))

    acc1, cnt1 = _seg1(t1, srcr, dstr, zrows, ones)
    cat = _seg2(acc1, cnt1, xr1, srcr, dstr, zrows)

    eye8 = jnp.eye(8, dtype=jnp.float32)
    outp = pl.pallas_call(
        _post_body,
        out_shape=jax.ShapeDtypeStruct((N // 8, 512), jnp.float32),
    )(cat.reshape((2 * NP + 2 * N) // 8, 128),
      jnp.kron(eye8, Wr2), jnp.tile(b2.reshape(1, 32), (1, 8)),
      jnp.kron(eye8, Wl2),
      jnp.kron(eye8, Wf1), jnp.tile(bf1.reshape(1, 64), (1, 8)),
      jnp.kron(eye8, Wf2), jnp.tile(bf2.reshape(1, 128), (1, 8)),
      jnp.kron(eye8, Wf3), jnp.tile(bf3.reshape(1, 64), (1, 8)))
    return outp.reshape(N, 64)
